# baseline (speedup 1.0000x reference)
import functools

import jax
import jax.numpy as jnp
from jax import lax
from jax.experimental import pallas as pl
from jax.experimental.pallas import tpu as pltpu
from jax.experimental.pallas import tpu_sc as plsc

B = 16384
EMB = 32
NTAB = 5
NC = 2
NS = 16
NW = NC * NS
BW = B // NW
NE = BW * EMB


def kernel(authdir, year, actor, rated, genre,
           W_authdir, W_year, W_actor, W_rated, W_genre):
    tabs = [W_authdir, W_year, W_actor, W_rated, W_genre]
    vocabs = [t.shape[0] for t in tabs]
    flats = [t.T.reshape(-1) for t in tabs]
    mesh = plsc.VectorSubcoreMesh(core_axis_name="c", subcore_axis_name="s",
                                  num_cores=NC, num_subcores=NS)

    @functools.partial(
        pl.kernel,
        mesh=mesh,
        out_type=jax.ShapeDtypeStruct((NTAB * EMB, B), jnp.float32),
        compiler_params=pltpu.CompilerParams(needs_layout_passes=False),
        scratch_types=(
            pltpu.VMEM((BW,), jnp.int32),
            pltpu.VMEM((NE,), jnp.int32),
            pltpu.VMEM((NE,), jnp.int32),
            pltpu.VMEM((NE,), jnp.float32),
            pltpu.VMEM((NE,), jnp.float32),
            pltpu.SemaphoreType.DMA,
            pltpu.SemaphoreType.DMA,
        ),
    )
    def body(a_i, y_i, ac_i, r_i, g_i, Fa, Fy, Fac, Fr, Fg, out,
             idx_v, ie0, ie1, d0, d1, sem_g, sem_w):
        wid = lax.axis_index("s") * NC + lax.axis_index("c")
        base = wid * BW
        idx_hbm = [a_i, y_i, ac_i, r_i, g_i]
        flat = [Fa, Fy, Fac, Fr, Fg]
        idx_e = [ie0, ie1]
        dst = [d0, d1]

        def build(c):
            pltpu.sync_copy(idx_hbm[c].at[pl.ds(base, BW)], idx_v)
            ie = idx_e[c % 2]
            V = vocabs[c]

            @plsc.parallel_loop(0, BW, 16)
            def _(j):
                v = idx_v[pl.ds(j, 16)]
                for e in range(EMB):
                    ie[pl.ds(e * BW + j, 16)] = v + e * V

        def start(c):
            return pltpu.async_copy(flat[c].at[idx_e[c % 2]], dst[c % 2],
                                    sem_g)

        def write_out(c):
            d = dst[c % 2]
            return [pltpu.async_copy(d.at[pl.ds(e * BW, BW)],
                                     out.at[c * EMB + e, pl.ds(base, BW)],
                                     sem_w)
                    for e in range(EMB)]

        writes = [[], []]
        build(0)
        stream = start(0)
        for c in range(1, NTAB):
            build(c)
            for w in writes[c % 2]:
                w.wait()
            stream.wait()
            stream = start(c)
            writes[(c - 1) % 2] = write_out(c - 1)
        stream.wait()
        for w in writes[(NTAB - 2) % 2] + write_out(NTAB - 1):
            w.wait()

    out_t = body(authdir, year, actor, rated, genre, *flats)
    return out_t.T
